# Initial kernel scaffold; baseline (speedup 1.0000x reference)
import jax
import jax.numpy as jnp
from jax.experimental import pallas as pl


def kernel(x, edge_index, W_emb, beta2, W_dec):
    raise NotImplementedError("write your pallas kernel here")


# speedup vs baseline: 7.7217x; 7.7217x over previous
import functools

import jax
import jax.numpy as jnp
from jax import lax
from jax.experimental import pallas as pl
from jax.experimental.pallas import tpu as pltpu
from jax.experimental.pallas import tpu_sc as plsc

N = 50000
D = 64
N_CLASSES = 40
NPAD = 50176
NC = 2
NS = 16
NW = NC * NS
B = 128
EPAD = 851968
BLK_S1 = EPAD // NW // B
BLK_S2 = EPAD // NS // B
DEN_SZ = 50176
DEN_CH = DEN_SZ // NS
HALF = 25000
ACC_H = 25088
ACC_CH = ACC_H // NS


def _proj_norm_body(xb, wb, hb, xnb):
    h = jnp.maximum(jnp.dot(xb[...], wb[...], preferred_element_type=jnp.float32), 0.0)
    s = jnp.sum(h * h, axis=1, keepdims=True)
    inv = 1.0 / jnp.maximum(jnp.sqrt(s), 1e-12)
    hb[...] = h
    xnb[...] = h * inv


def _proj_norm(x_pad, W_emb):
    return pl.pallas_call(
        _proj_norm_body,
        grid=(NPAD // 512,),
        in_specs=[
            pl.BlockSpec((512, D), lambda i: (i, 0)),
            pl.BlockSpec((D, D), lambda i: (0, 0)),
        ],
        out_specs=[
            pl.BlockSpec((512, D), lambda i: (i, 0)),
            pl.BlockSpec((512, D), lambda i: (i, 0)),
        ],
        out_shape=[
            jax.ShapeDtypeStruct((NPAD, D), jnp.float32),
            jax.ShapeDtypeStruct((NPAD, D), jnp.float32),
        ],
    )(x_pad, W_emb)


def _norm_only_body(ob, hb, xnb):
    h = ob[...]
    s = jnp.sum(h * h, axis=1, keepdims=True)
    inv = 1.0 / jnp.maximum(jnp.sqrt(s), 1e-12)
    hb[...] = h
    xnb[...] = h * inv


def _norm_only(o):
    return pl.pallas_call(
        _norm_only_body,
        grid=(NPAD // 512,),
        in_specs=[pl.BlockSpec((512, D), lambda i: (i, 0))],
        out_specs=[
            pl.BlockSpec((512, D), lambda i: (i, 0)),
            pl.BlockSpec((512, D), lambda i: (i, 0)),
        ],
        out_shape=[
            jax.ShapeDtypeStruct((NPAD, D), jnp.float32),
            jax.ShapeDtypeStruct((NPAD, D), jnp.float32),
        ],
    )(o)


def _den_merge_body(db, rb):
    a = db[0]
    b = db[1]
    rb[...] = 1.0 / (a + b + 1e-16)


def _den_merge(den2):
    return pl.pallas_call(
        _den_merge_body,
        out_shape=jax.ShapeDtypeStruct((DEN_SZ // 128, 128), jnp.float32),
    )(den2)


def _decode_body(ob, wb, yb):
    h = jnp.maximum(ob[...], 0.0)
    yb[...] = jnp.dot(h, wb[...], preferred_element_type=jnp.float32)


def _decode(o, W_dec):
    return pl.pallas_call(
        _decode_body,
        grid=(NPAD // 512,),
        in_specs=[
            pl.BlockSpec((512, D), lambda i: (i, 0)),
            pl.BlockSpec((D, N_CLASSES), lambda i: (0, 0)),
        ],
        out_specs=pl.BlockSpec((512, N_CLASSES), lambda i: (i, 0)),
        out_shape=jax.ShapeDtypeStruct((NPAD, N_CLASSES), jnp.float32),
    )(o, W_dec)


_MESH = plsc.VectorSubcoreMesh(
    core_axis_name="c", subcore_axis_name="s", num_cores=NC, num_subcores=NS)


@functools.partial(
    pl.kernel,
    out_type=[
        jax.ShapeDtypeStruct((EPAD,), jnp.float32),
        jax.ShapeDtypeStruct((NC * DEN_SZ,), jnp.float32),
    ],
    mesh=_MESH,
    compiler_params=pltpu.CompilerParams(
        needs_layout_passes=False, use_tc_tiling_on_sc=False),
    scratch_types=[
        pltpu.VMEM((B,), jnp.int32),
        pltpu.VMEM((B,), jnp.int32),
        pltpu.VMEM((B, D), jnp.float32),
        pltpu.VMEM((B, D), jnp.float32),
        pltpu.VMEM((B,), jnp.float32),
        pltpu.VMEM((16,), jnp.float32),
        pltpu.VMEM((DEN_CH,), jnp.float32),
        pltpu.VMEM_SHARED((DEN_SZ,), jnp.float32),
        pltpu.SemaphoreType.DMA,
        pltpu.SemaphoreType.DMA,
    ],
)
def _edge_weights(xn_hbm, src_hbm, dst_hbm, beta_hbm, w_hbm, den_hbm,
                  src_v, dst_v, rows_s, rows_d, w_v, beta_v, zden_v, den_sh,
                  sem1, sem2):
    c = lax.axis_index("c")
    s = lax.axis_index("s")
    wid = c * NS + s

    def _z(i, _):
        zden_v[pl.ds(i * 16, 16)] = jnp.zeros((16,), jnp.float32)
        return 0
    lax.fori_loop(0, DEN_CH // 16, _z, 0)
    pltpu.sync_copy(zden_v, den_sh.at[pl.ds(s * DEN_CH, DEN_CH)])
    plsc.subcore_barrier()

    pltpu.sync_copy(beta_hbm, beta_v)
    beta_vec = beta_v[...]

    def _block(i, _):
        base = wid * (EPAD // NW) + i * B
        pltpu.sync_copy(src_hbm.at[pl.ds(base, B)], src_v)
        pltpu.sync_copy(dst_hbm.at[pl.ds(base, B)], dst_v)
        cp1 = pltpu.async_copy(xn_hbm.at[src_v], rows_s, sem1)
        cp2 = pltpu.async_copy(xn_hbm.at[dst_v], rows_d, sem2)
        cp1.wait()
        cp2.wait()

        lanes = lax.iota(jnp.int32, 16)

        def _grp(g, _):
            tot = jnp.zeros((16,), jnp.float32)
            for b in range(16):
                e = g * 16 + b
                acc = rows_s[e, pl.ds(0, 16)] * rows_d[e, pl.ds(0, 16)]
                for k in range(1, 4):
                    acc = acc + (rows_s[e, pl.ds(k * 16, 16)]
                                 * rows_d[e, pl.ds(k * 16, 16)])
                onehot = jnp.where(lanes == b, 1.0, 0.0)
                tot = tot + jnp.sum(acc) * onehot
            w_v[pl.ds(g * 16, 16)] = jnp.exp(tot * beta_vec)
            return 0
        lax.fori_loop(0, B // 16, _grp, 0)

        pltpu.sync_copy(w_v, w_hbm.at[pl.ds(base, B)])
        pltpu.sync_copy(w_v, den_sh.at[dst_v], add=True)
        return 0
    lax.fori_loop(0, BLK_S1, _block, 0)

    plsc.subcore_barrier()
    pltpu.sync_copy(den_sh.at[pl.ds(s * DEN_CH, DEN_CH)],
                    den_hbm.at[pl.ds(c * DEN_SZ + s * DEN_CH, DEN_CH)])


@functools.partial(
    pl.kernel,
    out_type=jax.ShapeDtypeStruct((NPAD, D), jnp.float32),
    mesh=_MESH,
    compiler_params=pltpu.CompilerParams(
        needs_layout_passes=False, use_tc_tiling_on_sc=False),
    scratch_types=[
        pltpu.VMEM((B,), jnp.int32),
        pltpu.VMEM((B,), jnp.int32),
        pltpu.VMEM((B,), jnp.int32),
        pltpu.VMEM((B,), jnp.float32),
        pltpu.VMEM((B,), jnp.float32),
        pltpu.VMEM((B, D), jnp.float32),
        pltpu.VMEM_SHARED((ACC_H, D), jnp.float32),
        pltpu.SemaphoreType.DMA,
        pltpu.SemaphoreType.DMA,
    ],
)
def _aggregate(h_hbm, src_hbm, dst_hbm, w_hbm, rden_hbm, out_hbm,
               src_v, dst_v, dstl_v, w_v, rd_v, rows, acc_sh, sem1, sem2):
    c = lax.axis_index("c")
    s = lax.axis_index("s")
    lo = c * HALF

    def _z(r, _):
        for k in range(4):
            rows[r, pl.ds(k * 16, 16)] = jnp.zeros((16,), jnp.float32)
        return 0
    lax.fori_loop(0, B, _z, 0)
    for j in range(ACC_CH // B):
        pltpu.sync_copy(rows, acc_sh.at[pl.ds(s * ACC_CH + j * B, B)])
    rem = ACC_CH % B
    if rem:
        pltpu.sync_copy(rows.at[pl.ds(0, rem)],
                        acc_sh.at[pl.ds(s * ACC_CH + (ACC_CH // B) * B, rem)])
    plsc.subcore_barrier()

    def _block(i, _):
        base = s * (EPAD // NS) + i * B
        pltpu.sync_copy(dst_hbm.at[pl.ds(base, B)], dst_v)
        pltpu.sync_copy(src_hbm.at[pl.ds(base, B)], src_v)
        pltpu.sync_copy(w_hbm.at[pl.ds(base, B)], w_v)
        cp1 = pltpu.async_copy(rden_hbm.at[dst_v], rd_v, sem1)
        cp2 = pltpu.async_copy(h_hbm.at[src_v], rows, sem2)
        cp1.wait()
        cp2.wait()

        def _grp(g, _):
            dv = dst_v[pl.ds(g * 16, 16)]
            local = dv - lo
            valid = (local >= 0) & (local < HALF)
            dstl_v[pl.ds(g * 16, 16)] = jnp.where(valid, local, HALF)
            cv = w_v[pl.ds(g * 16, 16)] * rd_v[pl.ds(g * 16, 16)]
            for b in range(16):
                r = g * 16 + b
                csc = cv[b]
                for k in range(4):
                    rows[r, pl.ds(k * 16, 16)] = rows[r, pl.ds(k * 16, 16)] * csc
            return 0
        lax.fori_loop(0, B // 16, _grp, 0)

        pltpu.sync_copy(rows, acc_sh.at[dstl_v], add=True)
        return 0
    lax.fori_loop(0, BLK_S2, _block, 0)

    plsc.subcore_barrier()
    start = jnp.minimum(s * ACC_CH, HALF - ACC_CH)
    pltpu.sync_copy(acc_sh.at[pl.ds(start, ACC_CH)],
                    out_hbm.at[pl.ds(lo + start, ACC_CH)])


def _conv(h, xn, src, dst, beta_arr):
    w, den2 = _edge_weights(xn, src, dst, beta_arr)
    rden = _den_merge(den2.reshape(NC, DEN_SZ // 128, 128)).reshape(DEN_SZ)
    return _aggregate(h, src, dst, w, rden)


def kernel(x, edge_index, W_emb, beta2, W_dec):
    x_pad = jnp.pad(x, ((0, NPAD - N), (0, 0)))
    loop = jnp.arange(N, dtype=jnp.int32)
    pad_e = EPAD - (edge_index.shape[1] + N)
    src = jnp.concatenate([edge_index[0], loop,
                           jnp.zeros((pad_e,), jnp.int32)])
    dst = jnp.concatenate([edge_index[1], loop,
                           jnp.full((pad_e,), N, jnp.int32)])
    one = jnp.ones((16,), jnp.float32)
    beta_b = jnp.broadcast_to(beta2.astype(jnp.float32), (16,))

    h1, xn1 = _proj_norm(x_pad, W_emb)
    o1 = _conv(h1, xn1, src, dst, one)
    h2, xn2 = _norm_only(o1)
    o2 = _conv(h2, xn2, src, dst, beta_b)
    y = _decode(o2, W_dec)
    return y[:N]


# speedup vs baseline: 17.3143x; 2.2423x over previous
import functools

import jax
import jax.numpy as jnp
from jax import lax
from jax.experimental import pallas as pl
from jax.experimental.pallas import tpu as pltpu
from jax.experimental.pallas import tpu_sc as plsc

N = 50000
D = 64
DH = D // 2
N_CLASSES = 40
NPAD = 50176
NC = 2
NS = 16
NW = NC * NS
MB = 512
SUB = MB // 128
EPAD = 851968
MAC_A = EPAD // NW // MB
MAC_B = EPAD // NS // MB
ROWS_A = EPAD // NW // 128
ROWS_B = EPAD // NS // 128
DEN_SZ = 50176
DEN_CH = DEN_SZ // NS
ACC_CH = NPAD // NS


def _proj_norm_body(xb, wb, hsb, xnb):
    h = jnp.maximum(jnp.dot(xb[...], wb[...], preferred_element_type=jnp.float32), 0.0)
    s = jnp.sum(h * h, axis=1, keepdims=True)
    inv = 1.0 / jnp.maximum(jnp.sqrt(s), 1e-12)
    hsb[0] = h[:, :DH]
    hsb[1] = h[:, DH:]
    xnb[...] = h * inv


def _proj_norm(x_pad, W_emb):
    return pl.pallas_call(
        _proj_norm_body,
        grid=(NPAD // 512,),
        in_specs=[
            pl.BlockSpec((512, D), lambda i: (i, 0)),
            pl.BlockSpec((D, D), lambda i: (0, 0)),
        ],
        out_specs=[
            pl.BlockSpec((NC, 512, DH), lambda i: (0, i, 0)),
            pl.BlockSpec((512, D), lambda i: (i, 0)),
        ],
        out_shape=[
            jax.ShapeDtypeStruct((NC, NPAD, DH), jnp.float32),
            jax.ShapeDtypeStruct((NPAD, D), jnp.float32),
        ],
    )(x_pad, W_emb)


def _norm_only_body(ob, xnb):
    h = jnp.concatenate([ob[0], ob[1]], axis=1)
    s = jnp.sum(h * h, axis=1, keepdims=True)
    inv = 1.0 / jnp.maximum(jnp.sqrt(s), 1e-12)
    xnb[...] = h * inv


def _norm_only(o):
    return pl.pallas_call(
        _norm_only_body,
        grid=(NPAD // 512,),
        in_specs=[pl.BlockSpec((NC, 512, DH), lambda i: (0, i, 0))],
        out_specs=pl.BlockSpec((512, D), lambda i: (i, 0)),
        out_shape=jax.ShapeDtypeStruct((NPAD, D), jnp.float32),
    )(o)


def _den_merge_body(db, rb):
    a = db[0]
    b = db[1]
    rb[...] = 1.0 / (a + b + 1e-16)


def _den_merge(den2):
    return pl.pallas_call(
        _den_merge_body,
        out_shape=jax.ShapeDtypeStruct((DEN_SZ // 128, 128), jnp.float32),
    )(den2)


def _decode_body(ob, wb, yb):
    h = jnp.maximum(jnp.concatenate([ob[0], ob[1]], axis=1), 0.0)
    yb[...] = jnp.dot(h, wb[...], preferred_element_type=jnp.float32)


def _decode(o, W_dec):
    return pl.pallas_call(
        _decode_body,
        grid=(NPAD // 512,),
        in_specs=[
            pl.BlockSpec((NC, 512, DH), lambda i: (0, i, 0)),
            pl.BlockSpec((D, N_CLASSES), lambda i: (0, 0)),
        ],
        out_specs=pl.BlockSpec((512, N_CLASSES), lambda i: (i, 0)),
        out_shape=jax.ShapeDtypeStruct((NPAD, N_CLASSES), jnp.float32),
    )(o, W_dec)


_MESH = plsc.VectorSubcoreMesh(
    core_axis_name="c", subcore_axis_name="s", num_cores=NC, num_subcores=NS)


@functools.partial(
    pl.kernel,
    out_type=[
        jax.ShapeDtypeStruct((EPAD,), jnp.float32),
        jax.ShapeDtypeStruct((NC * DEN_SZ,), jnp.float32),
    ],
    mesh=_MESH,
    compiler_params=pltpu.CompilerParams(
        needs_layout_passes=False, use_tc_tiling_on_sc=False),
    scratch_types=[
        pltpu.VMEM((SUB, 128), jnp.int32),
        pltpu.VMEM((SUB, 128), jnp.int32),
        pltpu.VMEM((MB, D), jnp.float32),
        pltpu.VMEM((MB, D), jnp.float32),
        pltpu.VMEM((MB,), jnp.float32),
        pltpu.VMEM((16,), jnp.float32),
        pltpu.VMEM((DEN_CH,), jnp.float32),
        pltpu.VMEM_SHARED((DEN_SZ,), jnp.float32),
        pltpu.SemaphoreType.DMA,
        pltpu.SemaphoreType.DMA,
    ],
)
def _edge_weights(xn_hbm, src_hbm, dst_hbm, beta_hbm, w_hbm, den_hbm,
                  src_v, dst_v, rows_s, rows_d, w_v, beta_v, zden_v, den_sh,
                  sem1, sem2):
    c = lax.axis_index("c")
    s = lax.axis_index("s")
    wid = c * NS + s

    def _z(i, _):
        zden_v[pl.ds(i * 16, 16)] = jnp.zeros((16,), jnp.float32)
        return 0
    lax.fori_loop(0, DEN_CH // 16, _z, 0)
    pltpu.sync_copy(zden_v, den_sh.at[pl.ds(s * DEN_CH, DEN_CH)])
    plsc.subcore_barrier()

    pltpu.sync_copy(beta_hbm, beta_v)
    beta_vec = beta_v[...]
    lanes = lax.iota(jnp.int32, 16)

    def _mac(i, _):
        rb = wid * ROWS_A + i * SUB
        pltpu.sync_copy(src_hbm.at[pl.ds(rb, SUB)], src_v)
        pltpu.sync_copy(dst_hbm.at[pl.ds(rb, SUB)], dst_v)
        cps = [pltpu.async_copy(xn_hbm.at[src_v.at[j]],
                                rows_s.at[pl.ds(j * 128, 128)], sem1)
               for j in range(SUB)]
        cpd = [pltpu.async_copy(xn_hbm.at[dst_v.at[j]],
                                rows_d.at[pl.ds(j * 128, 128)], sem2)
               for j in range(SUB)]
        for cp in cps + cpd:
            cp.wait()

        def _grp(g, _):
            tot = jnp.zeros((16,), jnp.float32)
            for b in range(16):
                e = g * 16 + b
                acc = rows_s[e, pl.ds(0, 16)] * rows_d[e, pl.ds(0, 16)]
                for k in range(1, 4):
                    acc = acc + (rows_s[e, pl.ds(k * 16, 16)]
                                 * rows_d[e, pl.ds(k * 16, 16)])
                onehot = jnp.where(lanes == b, 1.0, 0.0)
                tot = tot + jnp.sum(acc) * onehot
            w_v[pl.ds(g * 16, 16)] = jnp.exp(tot * beta_vec)
            return 0
        lax.fori_loop(0, MB // 16, _grp, 0)

        pltpu.sync_copy(w_v, w_hbm.at[pl.ds(rb * 128, MB)])
        for j in range(SUB):
            pltpu.sync_copy(w_v.at[pl.ds(j * 128, 128)],
                            den_sh.at[dst_v.at[j]], add=True)
        return 0
    lax.fori_loop(0, MAC_A, _mac, 0)

    plsc.subcore_barrier()
    pltpu.sync_copy(den_sh.at[pl.ds(s * DEN_CH, DEN_CH)],
                    den_hbm.at[pl.ds(c * DEN_SZ + s * DEN_CH, DEN_CH)])


@functools.partial(
    pl.kernel,
    out_type=jax.ShapeDtypeStruct((NC, NPAD, DH), jnp.float32),
    mesh=_MESH,
    compiler_params=pltpu.CompilerParams(
        needs_layout_passes=False, use_tc_tiling_on_sc=False),
    scratch_types=[
        pltpu.VMEM((SUB, 128), jnp.int32),
        pltpu.VMEM((SUB, 128), jnp.int32),
        pltpu.VMEM((MB,), jnp.float32),
        pltpu.VMEM((MB,), jnp.float32),
        pltpu.VMEM((MB, DH), jnp.float32),
        pltpu.VMEM_SHARED((NPAD, DH), jnp.float32),
        pltpu.SemaphoreType.DMA,
        pltpu.SemaphoreType.DMA,
    ],
)
def _aggregate(h_hbm, src_hbm, dst_hbm, w_hbm, rden_hbm, out_hbm,
               src_v, dst_v, w_v, rd_v, rows, acc_sh, sem1, sem2):
    c = lax.axis_index("c")
    s = lax.axis_index("s")

    def _z(r, _):
        for k in range(DH // 16):
            rows[r, pl.ds(k * 16, 16)] = jnp.zeros((16,), jnp.float32)
        return 0
    lax.fori_loop(0, MB, _z, 0)
    for j in range(ACC_CH // MB):
        pltpu.sync_copy(rows, acc_sh.at[pl.ds(s * ACC_CH + j * MB, MB)])
    rem = ACC_CH % MB
    if rem:
        pltpu.sync_copy(rows.at[pl.ds(0, rem)],
                        acc_sh.at[pl.ds(s * ACC_CH + (ACC_CH // MB) * MB, rem)])
    plsc.subcore_barrier()

    def _mac(i, _):
        rb = s * ROWS_B + i * SUB
        pltpu.sync_copy(dst_hbm.at[pl.ds(rb, SUB)], dst_v)
        pltpu.sync_copy(src_hbm.at[pl.ds(rb, SUB)], src_v)
        pltpu.sync_copy(w_hbm.at[pl.ds(rb * 128, MB)], w_v)
        cpr = [pltpu.async_copy(rden_hbm.at[dst_v.at[j]],
                                rd_v.at[pl.ds(j * 128, 128)], sem1)
               for j in range(SUB)]
        cph = [pltpu.async_copy(h_hbm.at[c].at[src_v.at[j]],
                                rows.at[pl.ds(j * 128, 128)], sem2)
               for j in range(SUB)]
        for cp in cpr + cph:
            cp.wait()

        def _grp(g, _):
            cv = w_v[pl.ds(g * 16, 16)] * rd_v[pl.ds(g * 16, 16)]
            for b in range(16):
                r = g * 16 + b
                csc = cv[b]
                for k in range(DH // 16):
                    rows[r, pl.ds(k * 16, 16)] = rows[r, pl.ds(k * 16, 16)] * csc
            return 0
        lax.fori_loop(0, MB // 16, _grp, 0)

        for j in range(SUB):
            pltpu.sync_copy(rows.at[pl.ds(j * 128, 128)],
                            acc_sh.at[dst_v.at[j]], add=True)
        return 0
    lax.fori_loop(0, MAC_B, _mac, 0)

    plsc.subcore_barrier()
    pltpu.sync_copy(acc_sh.at[pl.ds(s * ACC_CH, ACC_CH)],
                    out_hbm.at[c, pl.ds(s * ACC_CH, ACC_CH)])


def _conv(hs, xn, src2d, dst2d, beta_arr):
    w, den2 = _edge_weights(xn, src2d, dst2d, beta_arr)
    rden = _den_merge(den2.reshape(NC, DEN_SZ // 128, 128)).reshape(DEN_SZ)
    return _aggregate(hs, src2d, dst2d, w, rden)


def kernel(x, edge_index, W_emb, beta2, W_dec):
    x_pad = jnp.pad(x, ((0, NPAD - N), (0, 0)))
    loop = jnp.arange(N, dtype=jnp.int32)
    pad_e = EPAD - (edge_index.shape[1] + N)
    src2d = jnp.concatenate([edge_index[0], loop,
                             jnp.zeros((pad_e,), jnp.int32)]).reshape(-1, 128)
    dst2d = jnp.concatenate([edge_index[1], loop,
                             jnp.full((pad_e,), N, jnp.int32)]).reshape(-1, 128)
    one = jnp.ones((16,), jnp.float32)
    beta_b = jnp.broadcast_to(beta2.astype(jnp.float32), (16,))

    hs1, xn1 = _proj_norm(x_pad, W_emb)
    o1 = _conv(hs1, xn1, src2d, dst2d, one)
    xn2 = _norm_only(o1)
    o2 = _conv(o1, xn2, src2d, dst2d, beta_b)
    y = _decode(o2, W_dec)
    return y[:N]


# speedup vs baseline: 22.1949x; 1.2819x over previous
import functools

import jax
import jax.numpy as jnp
from jax import lax
from jax.experimental import pallas as pl
from jax.experimental.pallas import tpu as pltpu
from jax.experimental.pallas import tpu_sc as plsc

N = 50000
D = 64
DH = D // 2
N_CLASSES = 40
NPAD = 50176
NC = 2
NS = 16
NW = NC * NS
MB = 512
SUB = MB // 128
EPAD = 851968
MBB = 256
SUBB = MBB // 128
MAC_A = EPAD // NW // MB
MAC_B = EPAD // NS // MBB
ROWS_A = EPAD // NW // 128
ROWS_B = EPAD // NS // 128
DEN_SZ = 50176
DEN_CH = DEN_SZ // NS
ACC_CH = NPAD // NS


def _proj_norm_body(xb, wb, hsb, xnb):
    h = jnp.maximum(jnp.dot(xb[...], wb[...], preferred_element_type=jnp.float32), 0.0)
    s = jnp.sum(h * h, axis=1, keepdims=True)
    inv = 1.0 / jnp.maximum(jnp.sqrt(s), 1e-12)
    hsb[0] = h[:, :DH]
    hsb[1] = h[:, DH:]
    xnb[...] = (h * inv).astype(jnp.bfloat16)


def _proj_norm(x_pad, W_emb):
    return pl.pallas_call(
        _proj_norm_body,
        grid=(NPAD // 512,),
        in_specs=[
            pl.BlockSpec((512, D), lambda i: (i, 0)),
            pl.BlockSpec((D, D), lambda i: (0, 0)),
        ],
        out_specs=[
            pl.BlockSpec((NC, 512, DH), lambda i: (0, i, 0)),
            pl.BlockSpec((512, D), lambda i: (i, 0)),
        ],
        out_shape=[
            jax.ShapeDtypeStruct((NC, NPAD, DH), jnp.float32),
            jax.ShapeDtypeStruct((NPAD, D), jnp.bfloat16),
        ],
    )(x_pad, W_emb)


def _norm_only_body(ob, xnb):
    h = jnp.concatenate([ob[0], ob[1]], axis=1)
    s = jnp.sum(h * h, axis=1, keepdims=True)
    inv = 1.0 / jnp.maximum(jnp.sqrt(s), 1e-12)
    xnb[...] = (h * inv).astype(jnp.bfloat16)


def _norm_only(o):
    return pl.pallas_call(
        _norm_only_body,
        grid=(NPAD // 512,),
        in_specs=[pl.BlockSpec((NC, 512, DH), lambda i: (0, i, 0))],
        out_specs=pl.BlockSpec((512, D), lambda i: (i, 0)),
        out_shape=jax.ShapeDtypeStruct((NPAD, D), jnp.bfloat16),
    )(o)


def _den_merge_body(db, rb):
    a = db[0]
    b = db[1]
    rb[...] = 1.0 / (a + b + 1e-16)


def _den_merge(den2):
    return pl.pallas_call(
        _den_merge_body,
        out_shape=jax.ShapeDtypeStruct((DEN_SZ // 128, 128), jnp.float32),
    )(den2)


def _decode_body(ob, wb, yb):
    h = jnp.maximum(jnp.concatenate([ob[0], ob[1]], axis=1), 0.0)
    yb[...] = jnp.dot(h, wb[...], preferred_element_type=jnp.float32)


def _decode(o, W_dec):
    return pl.pallas_call(
        _decode_body,
        grid=(NPAD // 512,),
        in_specs=[
            pl.BlockSpec((NC, 512, DH), lambda i: (0, i, 0)),
            pl.BlockSpec((D, N_CLASSES), lambda i: (0, 0)),
        ],
        out_specs=pl.BlockSpec((512, N_CLASSES), lambda i: (i, 0)),
        out_shape=jax.ShapeDtypeStruct((NPAD, N_CLASSES), jnp.float32),
    )(o, W_dec)


_MESH = plsc.VectorSubcoreMesh(
    core_axis_name="c", subcore_axis_name="s", num_cores=NC, num_subcores=NS)


@functools.partial(
    pl.kernel,
    out_type=[
        jax.ShapeDtypeStruct((EPAD,), jnp.float32),
        jax.ShapeDtypeStruct((NC * DEN_SZ,), jnp.float32),
    ],
    mesh=_MESH,
    compiler_params=pltpu.CompilerParams(
        needs_layout_passes=False, use_tc_tiling_on_sc=False),
    scratch_types=[
        pltpu.VMEM((SUB, 128), jnp.int32),
        pltpu.VMEM((SUB, 128), jnp.int32),
        pltpu.VMEM((MB, D), jnp.bfloat16),
        pltpu.VMEM((MB, D), jnp.bfloat16),
        pltpu.VMEM((SUB, 128), jnp.int32),
        pltpu.VMEM((SUB, 128), jnp.int32),
        pltpu.VMEM((MB, D), jnp.bfloat16),
        pltpu.VMEM((MB, D), jnp.bfloat16),
        pltpu.VMEM((MB,), jnp.float32),
        pltpu.VMEM((16,), jnp.float32),
        pltpu.VMEM((DEN_CH,), jnp.float32),
        pltpu.VMEM_SHARED((DEN_SZ,), jnp.float32),
        pltpu.SemaphoreType.DMA,
        pltpu.SemaphoreType.DMA,
        pltpu.SemaphoreType.DMA,
        pltpu.SemaphoreType.DMA,
    ],
)
def _edge_weights(xn_hbm, src_hbm, dst_hbm, beta_hbm, w_hbm, den_hbm,
                  src_v0, dst_v0, rs0, rd0, src_v1, dst_v1, rs1, rd1,
                  w_v, beta_v, zden_v, den_sh, ss0, sd0, ss1, sd1):
    c = lax.axis_index("c")
    s = lax.axis_index("s")
    wid = c * NS + s
    bufs = [(src_v0, dst_v0, rs0, rd0, ss0, sd0),
            (src_v1, dst_v1, rs1, rd1, ss1, sd1)]

    def _z(i, _):
        zden_v[pl.ds(i * 16, 16)] = jnp.zeros((16,), jnp.float32)
        return 0
    lax.fori_loop(0, DEN_CH // 16, _z, 0)
    pltpu.sync_copy(zden_v, den_sh.at[pl.ds(s * DEN_CH, DEN_CH)])
    plsc.subcore_barrier()

    pltpu.sync_copy(beta_hbm, beta_v)
    beta_vec = beta_v[...]
    lanes = lax.iota(jnp.int32, 16)

    def _fire(m, sv, dv, rs, rd, ss, sd):
        rb = wid * ROWS_A + m * SUB
        pltpu.sync_copy(src_hbm.at[pl.ds(rb, SUB)], sv)
        pltpu.sync_copy(dst_hbm.at[pl.ds(rb, SUB)], dv)
        for j in range(SUB):
            pltpu.async_copy(xn_hbm.at[sv.at[j]],
                             rs.at[pl.ds(j * 128, 128)], ss)
            pltpu.async_copy(xn_hbm.at[dv.at[j]],
                             rd.at[pl.ds(j * 128, 128)], sd)

    def _wait(sv, dv, rs, rd, ss, sd):
        for j in range(SUB):
            pltpu.make_async_copy(xn_hbm.at[sv.at[j]],
                                  rs.at[pl.ds(j * 128, 128)], ss).wait()
            pltpu.make_async_copy(xn_hbm.at[dv.at[j]],
                                  rd.at[pl.ds(j * 128, 128)], sd).wait()

    _fire(0, *bufs[0])

    def _pair(i, _):
        for p in (0, 1):
            m = i * 2 + p
            _fire(jnp.minimum(m + 1, MAC_A - 1), *bufs[1 - p])
            sv, dv, rs, rd, ss, sd = bufs[p]
            _wait(*bufs[p])

            def _grp(g, _):
                tot = jnp.zeros((16,), jnp.float32)
                for b in range(16):
                    e = g * 16 + b
                    acc = jnp.zeros((16,), jnp.float32)
                    for k in range(D // 32):
                        sa, sb = plsc.unpack(rs[e, pl.ds(k * 32, 32)],
                                             format=plsc.PackFormat.INTERLEAVED)
                        da, db = plsc.unpack(rd[e, pl.ds(k * 32, 32)],
                                             format=plsc.PackFormat.INTERLEAVED)
                        acc = acc + sa * da + sb * db
                    onehot = jnp.where(lanes == b, 1.0, 0.0)
                    tot = tot + jnp.sum(acc) * onehot
                w_v[pl.ds(g * 16, 16)] = jnp.exp(tot * beta_vec)
                return 0
            lax.fori_loop(0, MB // 16, _grp, 0)

            rb = wid * ROWS_A + m * SUB
            pltpu.sync_copy(w_v, w_hbm.at[pl.ds(rb * 128, MB)])
            for j in range(SUB):
                pltpu.sync_copy(w_v.at[pl.ds(j * 128, 128)],
                                den_sh.at[dv.at[j]], add=True)
        return 0
    lax.fori_loop(0, MAC_A // 2, _pair, 0)
    _wait(*bufs[0])

    plsc.subcore_barrier()
    pltpu.sync_copy(den_sh.at[pl.ds(s * DEN_CH, DEN_CH)],
                    den_hbm.at[pl.ds(c * DEN_SZ + s * DEN_CH, DEN_CH)])


@functools.partial(
    pl.kernel,
    out_type=jax.ShapeDtypeStruct((NC, NPAD, DH), jnp.float32),
    mesh=_MESH,
    compiler_params=pltpu.CompilerParams(
        needs_layout_passes=False, use_tc_tiling_on_sc=False),
    scratch_types=[
        pltpu.VMEM((SUBB, 128), jnp.int32),
        pltpu.VMEM((SUBB, 128), jnp.int32),
        pltpu.VMEM((MBB,), jnp.float32),
        pltpu.VMEM((MBB,), jnp.float32),
        pltpu.VMEM((MBB, DH), jnp.float32),
        pltpu.VMEM((SUBB, 128), jnp.int32),
        pltpu.VMEM((SUBB, 128), jnp.int32),
        pltpu.VMEM((MBB,), jnp.float32),
        pltpu.VMEM((MBB,), jnp.float32),
        pltpu.VMEM((MBB, DH), jnp.float32),
        pltpu.VMEM_SHARED((NPAD, DH), jnp.float32),
        pltpu.SemaphoreType.DMA,
        pltpu.SemaphoreType.DMA,
        pltpu.SemaphoreType.DMA,
        pltpu.SemaphoreType.DMA,
    ],
)
def _aggregate(h_hbm, src_hbm, dst_hbm, w_hbm, rden_hbm, out_hbm,
               sv0, dv0, wv0, rv0, rw0, sv1, dv1, wv1, rv1, rw1,
               acc_sh, sr0, sh0, sr1, sh1):
    c = lax.axis_index("c")
    s = lax.axis_index("s")
    bufs = [(sv0, dv0, wv0, rv0, rw0, sr0, sh0),
            (sv1, dv1, wv1, rv1, rw1, sr1, sh1)]

    def _z(r, _):
        for k in range(DH // 16):
            rw0[r, pl.ds(k * 16, 16)] = jnp.zeros((16,), jnp.float32)
        return 0
    lax.fori_loop(0, MBB, _z, 0)
    for j in range(ACC_CH // MBB):
        pltpu.sync_copy(rw0, acc_sh.at[pl.ds(s * ACC_CH + j * MBB, MBB)])
    rem = ACC_CH % MBB
    if rem:
        pltpu.sync_copy(rw0.at[pl.ds(0, rem)],
                        acc_sh.at[pl.ds(s * ACC_CH + (ACC_CH // MBB) * MBB, rem)])
    plsc.subcore_barrier()

    def _fire(m, sv, dv, wv, rv, rw, sr, sh):
        rb = s * ROWS_B + m * SUBB
        pltpu.sync_copy(dst_hbm.at[pl.ds(rb, SUBB)], dv)
        pltpu.sync_copy(src_hbm.at[pl.ds(rb, SUBB)], sv)
        pltpu.sync_copy(w_hbm.at[pl.ds(rb * 128, MBB)], wv)
        for j in range(SUBB):
            pltpu.async_copy(rden_hbm.at[dv.at[j]],
                             rv.at[pl.ds(j * 128, 128)], sr)
            pltpu.async_copy(h_hbm.at[c].at[sv.at[j]],
                             rw.at[pl.ds(j * 128, 128)], sh)

    def _wait(sv, dv, wv, rv, rw, sr, sh):
        for j in range(SUBB):
            pltpu.make_async_copy(rden_hbm.at[dv.at[j]],
                                  rv.at[pl.ds(j * 128, 128)], sr).wait()
            pltpu.make_async_copy(h_hbm.at[c].at[sv.at[j]],
                                  rw.at[pl.ds(j * 128, 128)], sh).wait()

    _fire(0, *bufs[0])

    def _pair(i, _):
        for p in (0, 1):
            m = i * 2 + p
            _fire(jnp.minimum(m + 1, MAC_B - 1), *bufs[1 - p])
            sv, dv, wv, rv, rw, sr, sh = bufs[p]
            _wait(*bufs[p])

            def _grp(g, _):
                cv = wv[pl.ds(g * 16, 16)] * rv[pl.ds(g * 16, 16)]
                for b in range(16):
                    r = g * 16 + b
                    csc = cv[b]
                    for k in range(DH // 16):
                        rw[r, pl.ds(k * 16, 16)] = rw[r, pl.ds(k * 16, 16)] * csc
                return 0
            lax.fori_loop(0, MBB // 16, _grp, 0)

            for j in range(SUBB):
                pltpu.sync_copy(rw.at[pl.ds(j * 128, 128)],
                                acc_sh.at[dv.at[j]], add=True)
        return 0
    lax.fori_loop(0, MAC_B // 2, _pair, 0)
    _wait(*bufs[0])

    plsc.subcore_barrier()
    pltpu.sync_copy(acc_sh.at[pl.ds(s * ACC_CH, ACC_CH)],
                    out_hbm.at[c, pl.ds(s * ACC_CH, ACC_CH)])


def _conv(hs, xn, src2d, dst2d, beta_arr):
    w, den2 = _edge_weights(xn, src2d, dst2d, beta_arr)
    rden = _den_merge(den2.reshape(NC, DEN_SZ // 128, 128)).reshape(DEN_SZ)
    return _aggregate(hs, src2d, dst2d, w, rden)


def kernel(x, edge_index, W_emb, beta2, W_dec):
    x_pad = jnp.pad(x, ((0, NPAD - N), (0, 0)))
    loop = jnp.arange(N, dtype=jnp.int32)
    pad_e = EPAD - (edge_index.shape[1] + N)
    src2d = jnp.concatenate([edge_index[0], loop,
                             jnp.zeros((pad_e,), jnp.int32)]).reshape(-1, 128)
    dst2d = jnp.concatenate([edge_index[1], loop,
                             jnp.full((pad_e,), N, jnp.int32)]).reshape(-1, 128)
    one = jnp.ones((16,), jnp.float32)
    beta_b = jnp.broadcast_to(beta2.astype(jnp.float32), (16,))

    hs1, xn1 = _proj_norm(x_pad, W_emb)
    o1 = _conv(hs1, xn1, src2d, dst2d, one)
    xn2 = _norm_only(o1)
    o2 = _conv(o1, xn2, src2d, dst2d, beta_b)
    y = _decode(o2, W_dec)
    return y[:N]


# speedup vs baseline: 22.4427x; 1.0112x over previous
import functools

import jax
import jax.numpy as jnp
from jax import lax
from jax.experimental import pallas as pl
from jax.experimental.pallas import tpu as pltpu
from jax.experimental.pallas import tpu_sc as plsc

N = 50000
D = 64
DH = D // 2
N_CLASSES = 40
NPAD = 50176
NC = 2
NS = 16
NW = NC * NS
MB = 512
SUB = MB // 128
EPAD = 851968
MBB = 256
SUBB = MBB // 128
MAC_A = EPAD // NW // MB
MAC_B = EPAD // NS // MBB
ROWS_A = EPAD // NW // 128
ROWS_B = EPAD // NS // 128
DEN_SZ = 50176
DEN_CH = DEN_SZ // NS
ACC_CH = NPAD // NS


def _proj_norm_body(xb, wb, hsb, xnb):
    h = jnp.maximum(jnp.dot(xb[...], wb[...], preferred_element_type=jnp.float32), 0.0)
    s = jnp.sum(h * h, axis=1, keepdims=True)
    inv = 1.0 / jnp.maximum(jnp.sqrt(s), 1e-12)
    hsb[0] = h[:, :DH]
    hsb[1] = h[:, DH:]
    xnb[...] = (h * inv).astype(jnp.bfloat16)


def _proj_norm(x_pad, W_emb):
    return pl.pallas_call(
        _proj_norm_body,
        grid=(NPAD // 512,),
        in_specs=[
            pl.BlockSpec((512, D), lambda i: (i, 0)),
            pl.BlockSpec((D, D), lambda i: (0, 0)),
        ],
        out_specs=[
            pl.BlockSpec((NC, 512, DH), lambda i: (0, i, 0)),
            pl.BlockSpec((512, D), lambda i: (i, 0)),
        ],
        out_shape=[
            jax.ShapeDtypeStruct((NC, NPAD, DH), jnp.float32),
            jax.ShapeDtypeStruct((NPAD, D), jnp.bfloat16),
        ],
    )(x_pad, W_emb)


def _norm_only_body(ob, xnb):
    h = jnp.concatenate([ob[0], ob[1]], axis=1)
    s = jnp.sum(h * h, axis=1, keepdims=True)
    inv = 1.0 / jnp.maximum(jnp.sqrt(s), 1e-12)
    xnb[...] = (h * inv).astype(jnp.bfloat16)


def _norm_only(o):
    return pl.pallas_call(
        _norm_only_body,
        grid=(NPAD // 512,),
        in_specs=[pl.BlockSpec((NC, 512, DH), lambda i: (0, i, 0))],
        out_specs=pl.BlockSpec((512, D), lambda i: (i, 0)),
        out_shape=jax.ShapeDtypeStruct((NPAD, D), jnp.bfloat16),
    )(o)


def _den_merge_body(db, rb):
    a = db[0]
    b = db[1]
    rb[...] = 1.0 / (a + b + 1e-16)


def _den_merge(den2):
    return pl.pallas_call(
        _den_merge_body,
        out_shape=jax.ShapeDtypeStruct((DEN_SZ // 128, 128), jnp.float32),
    )(den2)


def _decode_body(ob, wb, yb):
    h = jnp.maximum(jnp.concatenate([ob[0], ob[1]], axis=1), 0.0)
    yb[...] = jnp.dot(h, wb[...], preferred_element_type=jnp.float32)


def _decode(o, W_dec):
    return pl.pallas_call(
        _decode_body,
        grid=(NPAD // 512,),
        in_specs=[
            pl.BlockSpec((NC, 512, DH), lambda i: (0, i, 0)),
            pl.BlockSpec((D, N_CLASSES), lambda i: (0, 0)),
        ],
        out_specs=pl.BlockSpec((512, N_CLASSES), lambda i: (i, 0)),
        out_shape=jax.ShapeDtypeStruct((NPAD, N_CLASSES), jnp.float32),
    )(o, W_dec)


_MESH = plsc.VectorSubcoreMesh(
    core_axis_name="c", subcore_axis_name="s", num_cores=NC, num_subcores=NS)


@functools.partial(
    pl.kernel,
    out_type=[
        jax.ShapeDtypeStruct((EPAD,), jnp.float32),
        jax.ShapeDtypeStruct((NC * DEN_SZ,), jnp.float32),
    ],
    mesh=_MESH,
    compiler_params=pltpu.CompilerParams(
        needs_layout_passes=False, use_tc_tiling_on_sc=False),
    scratch_types=[
        pltpu.VMEM((SUB, 128), jnp.int32),
        pltpu.VMEM((SUB, 128), jnp.int32),
        pltpu.VMEM((MB, D), jnp.bfloat16),
        pltpu.VMEM((MB, D), jnp.bfloat16),
        pltpu.VMEM((SUB, 128), jnp.int32),
        pltpu.VMEM((SUB, 128), jnp.int32),
        pltpu.VMEM((MB, D), jnp.bfloat16),
        pltpu.VMEM((MB, D), jnp.bfloat16),
        pltpu.VMEM((MB,), jnp.float32),
        pltpu.VMEM((MB,), jnp.float32),
        pltpu.VMEM((16,), jnp.float32),
        pltpu.VMEM((DEN_CH,), jnp.float32),
        pltpu.VMEM_SHARED((DEN_SZ,), jnp.float32),
        pltpu.SemaphoreType.DMA,
        pltpu.SemaphoreType.DMA,
        pltpu.SemaphoreType.DMA,
        pltpu.SemaphoreType.DMA,
        pltpu.SemaphoreType.DMA,
        pltpu.SemaphoreType.DMA,
    ],
)
def _edge_weights(xn_hbm, src_hbm, dst_hbm, beta_hbm, w_hbm, den_hbm,
                  src_v0, dst_v0, rs0, rd0, src_v1, dst_v1, rs1, rd1,
                  w_v0, w_v1, beta_v, zden_v, den_sh,
                  ss0, sd0, ss1, sd1, sc0, sc1):
    c = lax.axis_index("c")
    s = lax.axis_index("s")
    wid = c * NS + s
    bufs = [(src_v0, dst_v0, rs0, rd0, w_v0, ss0, sd0, sc0),
            (src_v1, dst_v1, rs1, rd1, w_v1, ss1, sd1, sc1)]

    def _z(i, _):
        zden_v[pl.ds(i * 16, 16)] = jnp.zeros((16,), jnp.float32)
        return 0
    lax.fori_loop(0, DEN_CH // 16, _z, 0)
    pltpu.sync_copy(zden_v, den_sh.at[pl.ds(s * DEN_CH, DEN_CH)])
    plsc.subcore_barrier()

    pltpu.sync_copy(beta_hbm, beta_v)
    beta_vec = beta_v[...]
    lanes = lax.iota(jnp.int32, 16)

    def _fire(m, sv, dv, rs, rd, wv, ss, sd, sc):
        rb = wid * ROWS_A + m * SUB
        pltpu.sync_copy(src_hbm.at[pl.ds(rb, SUB)], sv)
        pltpu.sync_copy(dst_hbm.at[pl.ds(rb, SUB)], dv)
        for j in range(SUB):
            pltpu.async_copy(xn_hbm.at[sv.at[j]],
                             rs.at[pl.ds(j * 128, 128)], ss)
            pltpu.async_copy(xn_hbm.at[dv.at[j]],
                             rd.at[pl.ds(j * 128, 128)], sd)

    def _wait(sv, dv, rs, rd, wv, ss, sd, sc):
        for j in range(SUB):
            pltpu.make_async_copy(xn_hbm.at[sv.at[j]],
                                  rs.at[pl.ds(j * 128, 128)], ss).wait()
            pltpu.make_async_copy(xn_hbm.at[dv.at[j]],
                                  rd.at[pl.ds(j * 128, 128)], sd).wait()

    _fire(0, *bufs[0])

    def _pair(i, _):
        for p in (0, 1):
            m = i * 2 + p
            _fire(jnp.minimum(m + 1, MAC_A - 1), *bufs[1 - p])
            sv, dv, rs, rd, wv, ss, sd, sc = bufs[p]
            _wait(*bufs[p])

            def _grp(g, _):
                tot = jnp.zeros((16,), jnp.float32)
                for b in range(16):
                    e = g * 16 + b
                    acc = jnp.zeros((16,), jnp.float32)
                    for k in range(D // 32):
                        sa, sb = plsc.unpack(rs[e, pl.ds(k * 32, 32)],
                                             format=plsc.PackFormat.INTERLEAVED)
                        da, db = plsc.unpack(rd[e, pl.ds(k * 32, 32)],
                                             format=plsc.PackFormat.INTERLEAVED)
                        acc = acc + sa * da + sb * db
                    onehot = jnp.where(lanes == b, 1.0, 0.0)
                    tot = tot + jnp.sum(acc) * onehot
                wv[pl.ds(g * 16, 16)] = jnp.exp(tot * beta_vec)
                return 0
            lax.fori_loop(0, MB // 16, _grp, 0)

            rb = wid * ROWS_A + m * SUB
            pltpu.sync_copy(wv, w_hbm.at[pl.ds(rb * 128, MB)])
            for j in range(SUB):
                pltpu.sync_copy(wv.at[pl.ds(j * 128, 128)],
                                den_sh.at[dv.at[j]], add=True)
        return 0
    lax.fori_loop(0, MAC_A // 2, _pair, 0)
    _wait(*bufs[0])

    plsc.subcore_barrier()
    pltpu.sync_copy(den_sh.at[pl.ds(s * DEN_CH, DEN_CH)],
                    den_hbm.at[pl.ds(c * DEN_SZ + s * DEN_CH, DEN_CH)])


@functools.partial(
    pl.kernel,
    out_type=jax.ShapeDtypeStruct((NC, NPAD, DH), jnp.float32),
    mesh=_MESH,
    compiler_params=pltpu.CompilerParams(
        needs_layout_passes=False, use_tc_tiling_on_sc=False),
    scratch_types=[
        pltpu.VMEM((SUBB, 128), jnp.int32),
        pltpu.VMEM((SUBB, 128), jnp.int32),
        pltpu.VMEM((MBB,), jnp.float32),
        pltpu.VMEM((MBB,), jnp.float32),
        pltpu.VMEM((MBB, DH), jnp.float32),
        pltpu.VMEM((SUBB, 128), jnp.int32),
        pltpu.VMEM((SUBB, 128), jnp.int32),
        pltpu.VMEM((MBB,), jnp.float32),
        pltpu.VMEM((MBB,), jnp.float32),
        pltpu.VMEM((MBB, DH), jnp.float32),
        pltpu.VMEM_SHARED((NPAD, DH), jnp.float32),
        pltpu.SemaphoreType.DMA,
        pltpu.SemaphoreType.DMA,
        pltpu.SemaphoreType.DMA,
        pltpu.SemaphoreType.DMA,
        pltpu.SemaphoreType.DMA,
        pltpu.SemaphoreType.DMA,
    ],
)
def _aggregate(h_hbm, src_hbm, dst_hbm, w_hbm, rden_hbm, out_hbm,
               sv0, dv0, wv0, rv0, rw0, sv1, dv1, wv1, rv1, rw1,
               acc_sh, sr0, sh0, sc0, sr1, sh1, sc1):
    c = lax.axis_index("c")
    s = lax.axis_index("s")
    bufs = [(sv0, dv0, wv0, rv0, rw0, sr0, sh0, sc0),
            (sv1, dv1, wv1, rv1, rw1, sr1, sh1, sc1)]

    def _z(r, _):
        for k in range(DH // 16):
            rw0[r, pl.ds(k * 16, 16)] = jnp.zeros((16,), jnp.float32)
        return 0
    lax.fori_loop(0, MBB, _z, 0)
    for j in range(ACC_CH // MBB):
        pltpu.sync_copy(rw0, acc_sh.at[pl.ds(s * ACC_CH + j * MBB, MBB)])
    rem = ACC_CH % MBB
    if rem:
        pltpu.sync_copy(rw0.at[pl.ds(0, rem)],
                        acc_sh.at[pl.ds(s * ACC_CH + (ACC_CH // MBB) * MBB, rem)])
    plsc.subcore_barrier()

    def _fire(m, sv, dv, wv, rv, rw, sr, sh, sc):
        rb = s * ROWS_B + m * SUBB
        pltpu.sync_copy(dst_hbm.at[pl.ds(rb, SUBB)], dv)
        pltpu.sync_copy(src_hbm.at[pl.ds(rb, SUBB)], sv)
        pltpu.sync_copy(w_hbm.at[pl.ds(rb * 128, MBB)], wv)
        for j in range(SUBB):
            pltpu.async_copy(rden_hbm.at[dv.at[j]],
                             rv.at[pl.ds(j * 128, 128)], sr)
            pltpu.async_copy(h_hbm.at[c].at[sv.at[j]],
                             rw.at[pl.ds(j * 128, 128)], sh)

    def _wait(sv, dv, wv, rv, rw, sr, sh, sc):
        for j in range(SUBB):
            pltpu.make_async_copy(rden_hbm.at[dv.at[j]],
                                  rv.at[pl.ds(j * 128, 128)], sr).wait()
            pltpu.make_async_copy(h_hbm.at[c].at[sv.at[j]],
                                  rw.at[pl.ds(j * 128, 128)], sh).wait()

    def _post(sv, dv, wv, rv, rw, sr, sh, sc):
        for j in range(SUBB):
            pltpu.make_async_copy(rw.at[pl.ds(j * 128, 128)],
                                  acc_sh.at[dv.at[j]], sc).start(add=True)

    def _post_wait(sv, dv, wv, rv, rw, sr, sh, sc):
        for j in range(SUBB):
            pltpu.make_async_copy(rw.at[pl.ds(j * 128, 128)],
                                  acc_sh.at[dv.at[j]], sc).wait()

    _fire(0, *bufs[0])

    def _pair(i, _):
        for p in (0, 1):
            m = i * 2 + p
            if p == 0:
                @pl.when(i > 0)
                def _():
                    _post_wait(*bufs[1])
            else:
                _post_wait(*bufs[0])
            _fire(jnp.minimum(m + 1, MAC_B - 1), *bufs[1 - p])
            sv, dv, wv, rv, rw, sr, sh, sc = bufs[p]
            _wait(*bufs[p])

            def _grp(g, _):
                cv = wv[pl.ds(g * 16, 16)] * rv[pl.ds(g * 16, 16)]
                for b in range(16):
                    r = g * 16 + b
                    csc = cv[b]
                    for k in range(DH // 16):
                        rw[r, pl.ds(k * 16, 16)] = rw[r, pl.ds(k * 16, 16)] * csc
                return 0
            lax.fori_loop(0, MBB // 16, _grp, 0)

            _post(*bufs[p])
        return 0
    lax.fori_loop(0, MAC_B // 2, _pair, 0)
    _wait(*bufs[0])
    _post_wait(*bufs[1])

    plsc.subcore_barrier()
    pltpu.sync_copy(acc_sh.at[pl.ds(s * ACC_CH, ACC_CH)],
                    out_hbm.at[c, pl.ds(s * ACC_CH, ACC_CH)])


def _conv(hs, xn, src2d, dst2d, beta_arr):
    w, den2 = _edge_weights(xn, src2d, dst2d, beta_arr)
    rden = _den_merge(den2.reshape(NC, DEN_SZ // 128, 128)).reshape(DEN_SZ)
    return _aggregate(hs, src2d, dst2d, w, rden)


def kernel(x, edge_index, W_emb, beta2, W_dec):
    x_pad = jnp.pad(x, ((0, NPAD - N), (0, 0)))
    loop = jnp.arange(N, dtype=jnp.int32)
    pad_e = EPAD - (edge_index.shape[1] + N)
    src2d = jnp.concatenate([edge_index[0], loop,
                             jnp.zeros((pad_e,), jnp.int32)]).reshape(-1, 128)
    dst2d = jnp.concatenate([edge_index[1], loop,
                             jnp.full((pad_e,), N, jnp.int32)]).reshape(-1, 128)
    one = jnp.ones((16,), jnp.float32)
    beta_b = jnp.broadcast_to(beta2.astype(jnp.float32), (16,))

    hs1, xn1 = _proj_norm(x_pad, W_emb)
    o1 = _conv(hs1, xn1, src2d, dst2d, one)
    xn2 = _norm_only(o1)
    o2 = _conv(o1, xn2, src2d, dst2d, beta_b)
    y = _decode(o2, W_dec)
    return y[:N]


# speedup vs baseline: 27.6585x; 1.2324x over previous
import functools

import jax
import jax.numpy as jnp
from jax import lax
from jax.experimental import pallas as pl
from jax.experimental.pallas import tpu as pltpu
from jax.experimental.pallas import tpu_sc as plsc

N = 50000
D = 64
DH = D // 2
N_CLASSES = 40
NPAD = 50176
NC = 2
NS = 16
NW = NC * NS
MB = 512
SUB = MB // 128
EPAD = 851968
MBB = 256
SUBB = MBB // 128
MAC_A = EPAD // NW // MB
MAC_B = EPAD // NS // MBB
ROWS_A = EPAD // NW // 128
ROWS_B = EPAD // NS // 128
DEN_SZ = 50176
DEN_CH = DEN_SZ // NS
ACC_CH = NPAD // NS


def _proj_norm_body(xb, wb, hsb, xnb):
    h = jnp.maximum(jnp.dot(xb[...], wb[...], preferred_element_type=jnp.float32), 0.0)
    s = jnp.sum(h * h, axis=1, keepdims=True)
    inv = 1.0 / jnp.maximum(jnp.sqrt(s), 1e-12)
    hsb[0] = h[:, :DH]
    hsb[1] = h[:, DH:]
    xnb[...] = (h * inv).astype(jnp.bfloat16)


def _proj_norm(x_pad, W_emb):
    return pl.pallas_call(
        _proj_norm_body,
        grid=(NPAD // 512,),
        in_specs=[
            pl.BlockSpec((512, D), lambda i: (i, 0)),
            pl.BlockSpec((D, D), lambda i: (0, 0)),
        ],
        out_specs=[
            pl.BlockSpec((NC, 512, DH), lambda i: (0, i, 0)),
            pl.BlockSpec((512, D), lambda i: (i, 0)),
        ],
        out_shape=[
            jax.ShapeDtypeStruct((NC, NPAD, DH), jnp.float32),
            jax.ShapeDtypeStruct((NPAD, D), jnp.bfloat16),
        ],
    )(x_pad, W_emb)


def _norm_only_body(ob, xnb):
    h = jnp.concatenate([ob[0], ob[1]], axis=1)
    s = jnp.sum(h * h, axis=1, keepdims=True)
    inv = 1.0 / jnp.maximum(jnp.sqrt(s), 1e-12)
    xnb[...] = (h * inv).astype(jnp.bfloat16)


def _norm_only(o):
    return pl.pallas_call(
        _norm_only_body,
        grid=(NPAD // 512,),
        in_specs=[pl.BlockSpec((NC, 512, DH), lambda i: (0, i, 0))],
        out_specs=pl.BlockSpec((512, D), lambda i: (i, 0)),
        out_shape=jax.ShapeDtypeStruct((NPAD, D), jnp.bfloat16),
    )(o)


def _den_merge_body(db, rb):
    a = db[0]
    b = db[1]
    rb[...] = 1.0 / (a + b + 1e-16)


def _den_merge(den2):
    return pl.pallas_call(
        _den_merge_body,
        out_shape=jax.ShapeDtypeStruct((DEN_SZ // 128, 128), jnp.float32),
    )(den2)


def _decode_body(ob, wb, yb):
    h = jnp.maximum(jnp.concatenate([ob[0], ob[1]], axis=1), 0.0)
    yb[...] = jnp.dot(h, wb[...], preferred_element_type=jnp.float32)


def _decode(o, W_dec):
    return pl.pallas_call(
        _decode_body,
        grid=(NPAD // 512,),
        in_specs=[
            pl.BlockSpec((NC, 512, DH), lambda i: (0, i, 0)),
            pl.BlockSpec((D, N_CLASSES), lambda i: (0, 0)),
        ],
        out_specs=pl.BlockSpec((512, N_CLASSES), lambda i: (i, 0)),
        out_shape=jax.ShapeDtypeStruct((NPAD, N_CLASSES), jnp.float32),
    )(o, W_dec)


_MESH = plsc.VectorSubcoreMesh(
    core_axis_name="c", subcore_axis_name="s", num_cores=NC, num_subcores=NS)


@functools.partial(
    pl.kernel,
    out_type=[
        jax.ShapeDtypeStruct((EPAD,), jnp.float32),
        jax.ShapeDtypeStruct((NC * DEN_SZ,), jnp.float32),
    ],
    mesh=_MESH,
    compiler_params=pltpu.CompilerParams(
        needs_layout_passes=False, use_tc_tiling_on_sc=False),
    scratch_types=[
        pltpu.VMEM((SUB, 128), jnp.int32),
        pltpu.VMEM((SUB, 128), jnp.int32),
        pltpu.VMEM((MB, D), jnp.bfloat16),
        pltpu.VMEM((MB, D), jnp.bfloat16),
        pltpu.VMEM((SUB, 128), jnp.int32),
        pltpu.VMEM((SUB, 128), jnp.int32),
        pltpu.VMEM((MB, D), jnp.bfloat16),
        pltpu.VMEM((MB, D), jnp.bfloat16),
        pltpu.VMEM((MB,), jnp.float32),
        pltpu.VMEM((MB,), jnp.float32),
        pltpu.VMEM((16,), jnp.float32),
        pltpu.VMEM((DEN_CH,), jnp.float32),
        pltpu.VMEM_SHARED((DEN_SZ,), jnp.float32),
        pltpu.SemaphoreType.DMA,
        pltpu.SemaphoreType.DMA,
        pltpu.SemaphoreType.DMA,
        pltpu.SemaphoreType.DMA,
        pltpu.SemaphoreType.DMA,
        pltpu.SemaphoreType.DMA,
    ],
)
def _edge_weights(xn_hbm, src_hbm, dst_hbm, beta_hbm, w_hbm, den_hbm,
                  src_v0, dst_v0, rs0, rd0, src_v1, dst_v1, rs1, rd1,
                  w_v0, w_v1, beta_v, zden_v, den_sh,
                  ss0, sd0, ss1, sd1, sc0, sc1):
    c = lax.axis_index("c")
    s = lax.axis_index("s")
    wid = c * NS + s
    bufs = [(src_v0, dst_v0, rs0, rd0, w_v0, ss0, sd0, sc0),
            (src_v1, dst_v1, rs1, rd1, w_v1, ss1, sd1, sc1)]

    def _z(i, _):
        zden_v[pl.ds(i * 16, 16)] = jnp.zeros((16,), jnp.float32)
        return 0
    lax.fori_loop(0, DEN_CH // 16, _z, 0)
    pltpu.sync_copy(zden_v, den_sh.at[pl.ds(s * DEN_CH, DEN_CH)])
    plsc.subcore_barrier()

    pltpu.sync_copy(beta_hbm, beta_v)
    beta_vec = beta_v[...]
    lanes = lax.iota(jnp.int32, 16)

    def _fire(m, sv, dv, rs, rd, wv, ss, sd, sc):
        rb = wid * ROWS_A + m * SUB
        pltpu.sync_copy(src_hbm.at[pl.ds(rb, SUB)], sv)
        pltpu.sync_copy(dst_hbm.at[pl.ds(rb, SUB)], dv)
        for j in range(SUB):
            pltpu.async_copy(xn_hbm.at[sv.at[j]],
                             rs.at[pl.ds(j * 128, 128)], ss)
            pltpu.async_copy(xn_hbm.at[dv.at[j]],
                             rd.at[pl.ds(j * 128, 128)], sd)

    def _wait(sv, dv, rs, rd, wv, ss, sd, sc):
        for j in range(SUB):
            pltpu.make_async_copy(xn_hbm.at[sv.at[j]],
                                  rs.at[pl.ds(j * 128, 128)], ss).wait()
            pltpu.make_async_copy(xn_hbm.at[dv.at[j]],
                                  rd.at[pl.ds(j * 128, 128)], sd).wait()

    _fire(0, *bufs[0])

    def _pair(i, _):
        for p in (0, 1):
            m = i * 2 + p
            _fire(jnp.minimum(m + 1, MAC_A - 1), *bufs[1 - p])
            sv, dv, rs, rd, wv, ss, sd, sc = bufs[p]
            _wait(*bufs[p])

            def _grp(g, _):
                tot = jnp.zeros((16,), jnp.float32)
                for b in range(16):
                    e = g * 16 + b
                    acc = jnp.zeros((16,), jnp.float32)
                    for k in range(D // 32):
                        sa, sb = plsc.unpack(rs[e, pl.ds(k * 32, 32)],
                                             format=plsc.PackFormat.INTERLEAVED)
                        da, db = plsc.unpack(rd[e, pl.ds(k * 32, 32)],
                                             format=plsc.PackFormat.INTERLEAVED)
                        acc = acc + sa * da + sb * db
                    onehot = jnp.where(lanes == b, 1.0, 0.0)
                    tot = tot + jnp.sum(acc) * onehot
                wv[pl.ds(g * 16, 16)] = jnp.exp(tot * beta_vec)
                return 0
            lax.fori_loop(0, MB // 16, _grp, 0)

            rb = wid * ROWS_A + m * SUB
            pltpu.sync_copy(wv, w_hbm.at[pl.ds(rb * 128, MB)])
            for j in range(SUB):
                pltpu.sync_copy(wv.at[pl.ds(j * 128, 128)],
                                den_sh.at[dv.at[j]], add=True)
        return 0
    lax.fori_loop(0, MAC_A // 2, _pair, 0)
    _wait(*bufs[0])

    plsc.subcore_barrier()
    pltpu.sync_copy(den_sh.at[pl.ds(s * DEN_CH, DEN_CH)],
                    den_hbm.at[pl.ds(c * DEN_SZ + s * DEN_CH, DEN_CH)])


@functools.partial(
    pl.kernel,
    out_type=jax.ShapeDtypeStruct((NC, NPAD, DH), jnp.float32),
    mesh=_MESH,
    compiler_params=pltpu.CompilerParams(
        needs_layout_passes=False, use_tc_tiling_on_sc=False),
    scratch_types=[
        pltpu.VMEM((SUBB, 128), jnp.int32),
        pltpu.VMEM((SUBB, 128), jnp.int32),
        pltpu.VMEM((SUBB, 128), jnp.int32),
        pltpu.VMEM((MBB,), jnp.float32),
        pltpu.VMEM((MBB,), jnp.float32),
        pltpu.VMEM((MBB, DH), jnp.float32),
        pltpu.VMEM((SUBB, 128), jnp.int32),
        pltpu.VMEM((SUBB, 128), jnp.int32),
        pltpu.VMEM((SUBB, 128), jnp.int32),
        pltpu.VMEM((MBB,), jnp.float32),
        pltpu.VMEM((MBB,), jnp.float32),
        pltpu.VMEM((MBB, DH), jnp.float32),
        pltpu.VMEM_SHARED((NPAD, DH), jnp.float32),
        pltpu.SemaphoreType.DMA,
        pltpu.SemaphoreType.DMA,
        pltpu.SemaphoreType.DMA,
        pltpu.SemaphoreType.DMA,
        pltpu.SemaphoreType.DMA,
        pltpu.SemaphoreType.DMA,
        pltpu.SemaphoreType.DMA,
        pltpu.SemaphoreType.DMA,
    ],
)
def _aggregate(h_hbm, src_hbm, dst_hbm, w_hbm, rden_hbm, out_hbm,
               sv0, dv0, dsc0, wv0, rv0, rw0, sv1, dv1, dsc1, wv1, rv1, rw1,
               acc_sh, sl0, sr0, sh0, sc0, sl1, sr1, sh1, sc1):
    c = lax.axis_index("c")
    s = lax.axis_index("s")
    bufs = [(sv0, dv0, dsc0, wv0, rv0, rw0, sl0, sr0, sh0, sc0),
            (sv1, dv1, dsc1, wv1, rv1, rw1, sl1, sr1, sh1, sc1)]

    def _z(r, _):
        for k in range(DH // 16):
            rw0[r, pl.ds(k * 16, 16)] = jnp.zeros((16,), jnp.float32)
        return 0
    lax.fori_loop(0, MBB, _z, 0)
    for j in range(ACC_CH // MBB):
        pltpu.sync_copy(rw0, acc_sh.at[pl.ds(s * ACC_CH + j * MBB, MBB)])
    rem = ACC_CH % MBB
    if rem:
        pltpu.sync_copy(rw0.at[pl.ds(0, rem)],
                        acc_sh.at[pl.ds(s * ACC_CH + (ACC_CH // MBB) * MBB, rem)])
    plsc.subcore_barrier()

    def _idx(m, sv, dv, dsc, wv, rv, rw, sl, sr, sh, sc):
        rb = s * ROWS_B + m * SUBB
        pltpu.async_copy(dst_hbm.at[pl.ds(rb, SUBB)], dv, sl)
        pltpu.async_copy(src_hbm.at[pl.ds(rb, SUBB)], sv, sl)
        pltpu.async_copy(w_hbm.at[pl.ds(rb * 128, MBB)], wv, sl)

    def _idx_wait(sv, dv, dsc, wv, rv, rw, sl, sr, sh, sc):
        pltpu.make_async_copy(dst_hbm.at[pl.ds(0, SUBB)], dv, sl).wait()
        pltpu.make_async_copy(src_hbm.at[pl.ds(0, SUBB)], sv, sl).wait()
        pltpu.make_async_copy(w_hbm.at[pl.ds(0, MBB)], wv, sl).wait()

    def _gath(sv, dv, dsc, wv, rv, rw, sl, sr, sh, sc):
        for j in range(SUBB):
            pltpu.async_copy(rden_hbm.at[dv.at[j]],
                             rv.at[pl.ds(j * 128, 128)], sr)
            pltpu.async_copy(h_hbm.at[c].at[sv.at[j]],
                             rw.at[pl.ds(j * 128, 128)], sh)

    def _gath_wait(sv, dv, dsc, wv, rv, rw, sl, sr, sh, sc):
        for j in range(SUBB):
            pltpu.make_async_copy(rden_hbm.at[dv.at[j]],
                                  rv.at[pl.ds(j * 128, 128)], sr).wait()
            pltpu.make_async_copy(h_hbm.at[c].at[sv.at[j]],
                                  rw.at[pl.ds(j * 128, 128)], sh).wait()

    def _scat(sv, dv, dsc, wv, rv, rw, sl, sr, sh, sc):
        for j in range(SUBB):
            pltpu.make_async_copy(rw.at[pl.ds(j * 128, 128)],
                                  acc_sh.at[dsc.at[j]], sc).start(add=True)

    def _scat_wait(sv, dv, dsc, wv, rv, rw, sl, sr, sh, sc):
        for j in range(SUBB):
            pltpu.make_async_copy(rw.at[pl.ds(j * 128, 128)],
                                  acc_sh.at[dsc.at[j]], sc).wait()

    _idx(0, *bufs[0])
    _idx(1, *bufs[1])
    _idx_wait(*bufs[0])
    _gath(*bufs[0])

    def _pair(i, _):
        for p in (0, 1):
            m = i * 2 + p
            _idx_wait(*bufs[1 - p])
            if p == 0:
                @pl.when(i > 0)
                def _():
                    _scat_wait(*bufs[1])
            else:
                _scat_wait(*bufs[0])
            _gath(*bufs[1 - p])
            sv, dv, dsc, wv, rv, rw, sl, sr, sh, sc = bufs[p]
            _gath_wait(*bufs[p])

            def _grp(g, _):
                cv = wv[pl.ds(g * 16, 16)] * rv[pl.ds(g * 16, 16)]
                for b in range(16):
                    r = g * 16 + b
                    csc = cv[b]
                    for k in range(DH // 16):
                        rw[r, pl.ds(k * 16, 16)] = rw[r, pl.ds(k * 16, 16)] * csc
                return 0
            lax.fori_loop(0, MBB // 16, _grp, 0)
            for j in range(SUBB):
                for k in range(8):
                    dsc[j, pl.ds(k * 16, 16)] = dv[j, pl.ds(k * 16, 16)]

            _scat(*bufs[p])
            _idx(jnp.minimum(m + 2, MAC_B - 1), *bufs[p])
        return 0
    lax.fori_loop(0, MAC_B // 2, _pair, 0)
    _gath_wait(*bufs[0])
    _idx_wait(*bufs[1])
    _scat_wait(*bufs[1])

    plsc.subcore_barrier()
    pltpu.sync_copy(acc_sh.at[pl.ds(s * ACC_CH, ACC_CH)],
                    out_hbm.at[c, pl.ds(s * ACC_CH, ACC_CH)])


def _conv(hs, xn, src2d, dst2d, beta_arr):
    w, den2 = _edge_weights(xn, src2d, dst2d, beta_arr)
    rden = _den_merge(den2.reshape(NC, DEN_SZ // 128, 128)).reshape(DEN_SZ)
    return _aggregate(hs, src2d, dst2d, w, rden)


def kernel(x, edge_index, W_emb, beta2, W_dec):
    x_pad = jnp.pad(x, ((0, NPAD - N), (0, 0)))
    loop = jnp.arange(N, dtype=jnp.int32)
    pad_e = EPAD - (edge_index.shape[1] + N)
    src2d = jnp.concatenate([edge_index[0], loop,
                             jnp.zeros((pad_e,), jnp.int32)]).reshape(-1, 128)
    dst2d = jnp.concatenate([edge_index[1], loop,
                             jnp.full((pad_e,), N, jnp.int32)]).reshape(-1, 128)
    one = jnp.ones((16,), jnp.float32)
    beta_b = jnp.broadcast_to(beta2.astype(jnp.float32), (16,))

    hs1, xn1 = _proj_norm(x_pad, W_emb)
    o1 = _conv(hs1, xn1, src2d, dst2d, one)
    xn2 = _norm_only(o1)
    o2 = _conv(o1, xn2, src2d, dst2d, beta_b)
    y = _decode(o2, W_dec)
    return y[:N]


# speedup vs baseline: 28.8387x; 1.0427x over previous
import functools

import jax
import jax.numpy as jnp
from jax import lax
from jax.experimental import pallas as pl
from jax.experimental.pallas import tpu as pltpu
from jax.experimental.pallas import tpu_sc as plsc

N = 50000
D = 64
DH = D // 2
N_CLASSES = 40
NPAD = 50176
NC = 2
NS = 16
NW = NC * NS
MB = 512
SUB = MB // 128
EPAD = 851968
MBB = 256
SUBB = MBB // 128
MAC_A = EPAD // NW // MB
MAC_B = EPAD // NS // MBB
ROWS_A = EPAD // NW // 128
ROWS_B = EPAD // NS // 128
DEN_SZ = 50176
DEN_CH = DEN_SZ // NS
ACC_CH = NPAD // NS


def _proj_norm_body(xb, wb, hsb, xnb):
    h = jnp.maximum(jnp.dot(xb[...], wb[...], preferred_element_type=jnp.float32), 0.0)
    s = jnp.sum(h * h, axis=1, keepdims=True)
    inv = 1.0 / jnp.maximum(jnp.sqrt(s), 1e-12)
    hsb[0] = h[:, :DH]
    hsb[1] = h[:, DH:]
    xnb[...] = (h * inv).astype(jnp.bfloat16)


def _proj_norm(x_pad, W_emb):
    return pl.pallas_call(
        _proj_norm_body,
        grid=(NPAD // 512,),
        in_specs=[
            pl.BlockSpec((512, D), lambda i: (i, 0)),
            pl.BlockSpec((D, D), lambda i: (0, 0)),
        ],
        out_specs=[
            pl.BlockSpec((NC, 512, DH), lambda i: (0, i, 0)),
            pl.BlockSpec((512, D), lambda i: (i, 0)),
        ],
        out_shape=[
            jax.ShapeDtypeStruct((NC, NPAD, DH), jnp.float32),
            jax.ShapeDtypeStruct((NPAD, D), jnp.bfloat16),
        ],
    )(x_pad, W_emb)


def _norm_only_body(ob, xnb):
    h = jnp.concatenate([ob[0], ob[1]], axis=1)
    s = jnp.sum(h * h, axis=1, keepdims=True)
    inv = 1.0 / jnp.maximum(jnp.sqrt(s), 1e-12)
    xnb[...] = (h * inv).astype(jnp.bfloat16)


def _norm_only(o):
    return pl.pallas_call(
        _norm_only_body,
        grid=(NPAD // 512,),
        in_specs=[pl.BlockSpec((NC, 512, DH), lambda i: (0, i, 0))],
        out_specs=pl.BlockSpec((512, D), lambda i: (i, 0)),
        out_shape=jax.ShapeDtypeStruct((NPAD, D), jnp.bfloat16),
    )(o)


def _den_merge_body(db, rb):
    a = db[0]
    b = db[1]
    rb[...] = 1.0 / (a + b + 1e-16)


def _den_merge(den2):
    return pl.pallas_call(
        _den_merge_body,
        out_shape=jax.ShapeDtypeStruct((DEN_SZ // 128, 128), jnp.float32),
    )(den2)


def _decode_body(ob, wb, yb):
    h = jnp.maximum(jnp.concatenate([ob[0], ob[1]], axis=1), 0.0)
    yb[...] = jnp.dot(h, wb[...], preferred_element_type=jnp.float32)


def _decode(o, W_dec):
    return pl.pallas_call(
        _decode_body,
        grid=(NPAD // 512,),
        in_specs=[
            pl.BlockSpec((NC, 512, DH), lambda i: (0, i, 0)),
            pl.BlockSpec((D, N_CLASSES), lambda i: (0, 0)),
        ],
        out_specs=pl.BlockSpec((512, N_CLASSES), lambda i: (i, 0)),
        out_shape=jax.ShapeDtypeStruct((NPAD, N_CLASSES), jnp.float32),
    )(o, W_dec)


_MESH = plsc.VectorSubcoreMesh(
    core_axis_name="c", subcore_axis_name="s", num_cores=NC, num_subcores=NS)


@functools.partial(
    pl.kernel,
    out_type=[
        jax.ShapeDtypeStruct((EPAD,), jnp.float32),
        jax.ShapeDtypeStruct((NC * DEN_SZ,), jnp.float32),
    ],
    mesh=_MESH,
    compiler_params=pltpu.CompilerParams(
        needs_layout_passes=False, use_tc_tiling_on_sc=False),
    scratch_types=[
        pltpu.VMEM((SUB, 128), jnp.int32),
        pltpu.VMEM((SUB, 128), jnp.int32),
        pltpu.VMEM((SUB, 128), jnp.int32),
        pltpu.VMEM((MB, D), jnp.bfloat16),
        pltpu.VMEM((MB, D), jnp.bfloat16),
        pltpu.VMEM((MB,), jnp.float32),
        pltpu.VMEM((SUB, 128), jnp.int32),
        pltpu.VMEM((SUB, 128), jnp.int32),
        pltpu.VMEM((SUB, 128), jnp.int32),
        pltpu.VMEM((MB, D), jnp.bfloat16),
        pltpu.VMEM((MB, D), jnp.bfloat16),
        pltpu.VMEM((MB,), jnp.float32),
        pltpu.VMEM((16,), jnp.float32),
        pltpu.VMEM((DEN_CH,), jnp.float32),
        pltpu.VMEM_SHARED((DEN_SZ,), jnp.float32),
        pltpu.SemaphoreType.DMA,
        pltpu.SemaphoreType.DMA,
        pltpu.SemaphoreType.DMA,
        pltpu.SemaphoreType.DMA,
        pltpu.SemaphoreType.DMA,
        pltpu.SemaphoreType.DMA,
        pltpu.SemaphoreType.DMA,
        pltpu.SemaphoreType.DMA,
        pltpu.SemaphoreType.DMA,
        pltpu.SemaphoreType.DMA,
    ],
)
def _edge_weights(xn_hbm, src_hbm, dst_hbm, beta_hbm, w_hbm, den_hbm,
                  sv0, dv0, dsc0, rs0, rd0, wv0, sv1, dv1, dsc1, rs1, rd1, wv1,
                  beta_v, zden_v, den_sh,
                  sl0, ss0, sd0, sw0, sc0, sl1, ss1, sd1, sw1, sc1):
    c = lax.axis_index("c")
    s = lax.axis_index("s")
    wid = c * NS + s
    bufs = [(sv0, dv0, dsc0, rs0, rd0, wv0, sl0, ss0, sd0, sw0, sc0),
            (sv1, dv1, dsc1, rs1, rd1, wv1, sl1, ss1, sd1, sw1, sc1)]

    def _z(i, _):
        zden_v[pl.ds(i * 16, 16)] = jnp.zeros((16,), jnp.float32)
        return 0
    lax.fori_loop(0, DEN_CH // 16, _z, 0)
    pltpu.sync_copy(zden_v, den_sh.at[pl.ds(s * DEN_CH, DEN_CH)])
    plsc.subcore_barrier()

    pltpu.sync_copy(beta_hbm, beta_v)
    beta_vec = beta_v[...]
    lanes = lax.iota(jnp.int32, 16)

    def _idx(m, sv, dv, dsc, rs, rd, wv, sl, ss, sd, sw, sc):
        rb = wid * ROWS_A + m * SUB
        pltpu.async_copy(src_hbm.at[pl.ds(rb, SUB)], sv, sl)
        pltpu.async_copy(dst_hbm.at[pl.ds(rb, SUB)], dv, sl)

    def _idx_wait(sv, dv, dsc, rs, rd, wv, sl, ss, sd, sw, sc):
        pltpu.make_async_copy(src_hbm.at[pl.ds(0, SUB)], sv, sl).wait()
        pltpu.make_async_copy(dst_hbm.at[pl.ds(0, SUB)], dv, sl).wait()

    def _gath(sv, dv, dsc, rs, rd, wv, sl, ss, sd, sw, sc):
        for j in range(SUB):
            pltpu.async_copy(xn_hbm.at[sv.at[j]],
                             rs.at[pl.ds(j * 128, 128)], ss)
            pltpu.async_copy(xn_hbm.at[dv.at[j]],
                             rd.at[pl.ds(j * 128, 128)], sd)

    def _gath_wait(sv, dv, dsc, rs, rd, wv, sl, ss, sd, sw, sc):
        for j in range(SUB):
            pltpu.make_async_copy(xn_hbm.at[sv.at[j]],
                                  rs.at[pl.ds(j * 128, 128)], ss).wait()
            pltpu.make_async_copy(xn_hbm.at[dv.at[j]],
                                  rd.at[pl.ds(j * 128, 128)], sd).wait()

    def _post(m, sv, dv, dsc, rs, rd, wv, sl, ss, sd, sw, sc):
        rb = wid * ROWS_A + m * SUB
        pltpu.make_async_copy(wv, w_hbm.at[pl.ds(rb * 128, MB)], sw).start()
        for j in range(SUB):
            pltpu.make_async_copy(wv.at[pl.ds(j * 128, 128)],
                                  den_sh.at[dsc.at[j]], sc).start(add=True)

    def _post_wait(sv, dv, dsc, rs, rd, wv, sl, ss, sd, sw, sc):
        pltpu.make_async_copy(wv, w_hbm.at[pl.ds(0, MB)], sw).wait()
        for j in range(SUB):
            pltpu.make_async_copy(wv.at[pl.ds(j * 128, 128)],
                                  den_sh.at[dsc.at[j]], sc).wait()

    _idx(0, *bufs[0])
    _idx(1, *bufs[1])
    _idx_wait(*bufs[0])
    _gath(*bufs[0])

    def _pair(i, _):
        for p in (0, 1):
            m = i * 2 + p
            _idx_wait(*bufs[1 - p])
            if p == 0:
                @pl.when(i > 0)
                def _():
                    _post_wait(*bufs[1])
            else:
                _post_wait(*bufs[0])
            _gath(*bufs[1 - p])
            sv, dv, dsc, rs, rd, wv, sl, ss, sd, sw, sc = bufs[p]
            _gath_wait(*bufs[p])

            def _grp(g, _):
                tot = jnp.zeros((16,), jnp.float32)
                for b in range(16):
                    e = g * 16 + b
                    acc = jnp.zeros((16,), jnp.float32)
                    for k in range(D // 32):
                        sa, sb = plsc.unpack(rs[e, pl.ds(k * 32, 32)],
                                             format=plsc.PackFormat.INTERLEAVED)
                        da, db = plsc.unpack(rd[e, pl.ds(k * 32, 32)],
                                             format=plsc.PackFormat.INTERLEAVED)
                        acc = acc + sa * da + sb * db
                    onehot = jnp.where(lanes == b, 1.0, 0.0)
                    tot = tot + jnp.sum(acc) * onehot
                wv[pl.ds(g * 16, 16)] = jnp.exp(tot * beta_vec)
                return 0
            lax.fori_loop(0, MB // 16, _grp, 0)
            for j in range(SUB):
                for k in range(8):
                    dsc[j, pl.ds(k * 16, 16)] = dv[j, pl.ds(k * 16, 16)]

            _post(m, *bufs[p])
            _idx(jnp.minimum(m + 2, MAC_A - 1), *bufs[p])
        return 0
    lax.fori_loop(0, MAC_A // 2, _pair, 0)
    _gath_wait(*bufs[0])
    _idx_wait(*bufs[1])
    _post_wait(*bufs[1])

    plsc.subcore_barrier()
    pltpu.sync_copy(den_sh.at[pl.ds(s * DEN_CH, DEN_CH)],
                    den_hbm.at[pl.ds(c * DEN_SZ + s * DEN_CH, DEN_CH)])


@functools.partial(
    pl.kernel,
    out_type=jax.ShapeDtypeStruct((NC, NPAD, DH), jnp.float32),
    mesh=_MESH,
    compiler_params=pltpu.CompilerParams(
        needs_layout_passes=False, use_tc_tiling_on_sc=False),
    scratch_types=[
        pltpu.VMEM((SUBB, 128), jnp.int32),
        pltpu.VMEM((SUBB, 128), jnp.int32),
        pltpu.VMEM((SUBB, 128), jnp.int32),
        pltpu.VMEM((MBB,), jnp.float32),
        pltpu.VMEM((MBB,), jnp.float32),
        pltpu.VMEM((MBB, DH), jnp.float32),
        pltpu.VMEM((SUBB, 128), jnp.int32),
        pltpu.VMEM((SUBB, 128), jnp.int32),
        pltpu.VMEM((SUBB, 128), jnp.int32),
        pltpu.VMEM((MBB,), jnp.float32),
        pltpu.VMEM((MBB,), jnp.float32),
        pltpu.VMEM((MBB, DH), jnp.float32),
        pltpu.VMEM_SHARED((NPAD, DH), jnp.float32),
        pltpu.SemaphoreType.DMA,
        pltpu.SemaphoreType.DMA,
        pltpu.SemaphoreType.DMA,
        pltpu.SemaphoreType.DMA,
        pltpu.SemaphoreType.DMA,
        pltpu.SemaphoreType.DMA,
        pltpu.SemaphoreType.DMA,
        pltpu.SemaphoreType.DMA,
    ],
)
def _aggregate(h_hbm, src_hbm, dst_hbm, w_hbm, rden_hbm, out_hbm,
               sv0, dv0, dsc0, wv0, rv0, rw0, sv1, dv1, dsc1, wv1, rv1, rw1,
               acc_sh, sl0, sr0, sh0, sc0, sl1, sr1, sh1, sc1):
    c = lax.axis_index("c")
    s = lax.axis_index("s")
    bufs = [(sv0, dv0, dsc0, wv0, rv0, rw0, sl0, sr0, sh0, sc0),
            (sv1, dv1, dsc1, wv1, rv1, rw1, sl1, sr1, sh1, sc1)]

    def _z(r, _):
        for k in range(DH // 16):
            rw0[r, pl.ds(k * 16, 16)] = jnp.zeros((16,), jnp.float32)
        return 0
    lax.fori_loop(0, MBB, _z, 0)
    for j in range(ACC_CH // MBB):
        pltpu.sync_copy(rw0, acc_sh.at[pl.ds(s * ACC_CH + j * MBB, MBB)])
    rem = ACC_CH % MBB
    if rem:
        pltpu.sync_copy(rw0.at[pl.ds(0, rem)],
                        acc_sh.at[pl.ds(s * ACC_CH + (ACC_CH // MBB) * MBB, rem)])
    plsc.subcore_barrier()

    def _idx(m, sv, dv, dsc, wv, rv, rw, sl, sr, sh, sc):
        rb = s * ROWS_B + m * SUBB
        pltpu.async_copy(dst_hbm.at[pl.ds(rb, SUBB)], dv, sl)
        pltpu.async_copy(src_hbm.at[pl.ds(rb, SUBB)], sv, sl)
        pltpu.async_copy(w_hbm.at[pl.ds(rb * 128, MBB)], wv, sl)

    def _idx_wait(sv, dv, dsc, wv, rv, rw, sl, sr, sh, sc):
        pltpu.make_async_copy(dst_hbm.at[pl.ds(0, SUBB)], dv, sl).wait()
        pltpu.make_async_copy(src_hbm.at[pl.ds(0, SUBB)], sv, sl).wait()
        pltpu.make_async_copy(w_hbm.at[pl.ds(0, MBB)], wv, sl).wait()

    def _gath(sv, dv, dsc, wv, rv, rw, sl, sr, sh, sc):
        for j in range(SUBB):
            pltpu.async_copy(rden_hbm.at[dv.at[j]],
                             rv.at[pl.ds(j * 128, 128)], sr)
            pltpu.async_copy(h_hbm.at[c].at[sv.at[j]],
                             rw.at[pl.ds(j * 128, 128)], sh)

    def _gath_wait(sv, dv, dsc, wv, rv, rw, sl, sr, sh, sc):
        for j in range(SUBB):
            pltpu.make_async_copy(rden_hbm.at[dv.at[j]],
                                  rv.at[pl.ds(j * 128, 128)], sr).wait()
            pltpu.make_async_copy(h_hbm.at[c].at[sv.at[j]],
                                  rw.at[pl.ds(j * 128, 128)], sh).wait()

    def _scat(sv, dv, dsc, wv, rv, rw, sl, sr, sh, sc):
        for j in range(SUBB):
            pltpu.make_async_copy(rw.at[pl.ds(j * 128, 128)],
                                  acc_sh.at[dsc.at[j]], sc).start(add=True)

    def _scat_wait(sv, dv, dsc, wv, rv, rw, sl, sr, sh, sc):
        for j in range(SUBB):
            pltpu.make_async_copy(rw.at[pl.ds(j * 128, 128)],
                                  acc_sh.at[dsc.at[j]], sc).wait()

    _idx(0, *bufs[0])
    _idx(1, *bufs[1])
    _idx_wait(*bufs[0])
    _gath(*bufs[0])

    def _pair(i, _):
        for p in (0, 1):
            m = i * 2 + p
            _idx_wait(*bufs[1 - p])
            if p == 0:
                @pl.when(i > 0)
                def _():
                    _scat_wait(*bufs[1])
            else:
                _scat_wait(*bufs[0])
            _gath(*bufs[1 - p])
            sv, dv, dsc, wv, rv, rw, sl, sr, sh, sc = bufs[p]
            _gath_wait(*bufs[p])

            def _grp(g, _):
                cv = wv[pl.ds(g * 16, 16)] * rv[pl.ds(g * 16, 16)]
                for b in range(16):
                    r = g * 16 + b
                    csc = cv[b]
                    for k in range(DH // 16):
                        rw[r, pl.ds(k * 16, 16)] = rw[r, pl.ds(k * 16, 16)] * csc
                return 0
            lax.fori_loop(0, MBB // 16, _grp, 0)
            for j in range(SUBB):
                for k in range(8):
                    dsc[j, pl.ds(k * 16, 16)] = dv[j, pl.ds(k * 16, 16)]

            _scat(*bufs[p])
            _idx(jnp.minimum(m + 2, MAC_B - 1), *bufs[p])
        return 0
    lax.fori_loop(0, MAC_B // 2, _pair, 0)
    _gath_wait(*bufs[0])
    _idx_wait(*bufs[1])
    _scat_wait(*bufs[1])

    plsc.subcore_barrier()
    pltpu.sync_copy(acc_sh.at[pl.ds(s * ACC_CH, ACC_CH)],
                    out_hbm.at[c, pl.ds(s * ACC_CH, ACC_CH)])


def _conv(hs, xn, src2d, dst2d, beta_arr):
    w, den2 = _edge_weights(xn, src2d, dst2d, beta_arr)
    rden = _den_merge(den2.reshape(NC, DEN_SZ // 128, 128)).reshape(DEN_SZ)
    return _aggregate(hs, src2d, dst2d, w, rden)


def kernel(x, edge_index, W_emb, beta2, W_dec):
    x_pad = jnp.pad(x, ((0, NPAD - N), (0, 0)))
    loop = jnp.arange(N, dtype=jnp.int32)
    pad_e = EPAD - (edge_index.shape[1] + N)
    src2d = jnp.concatenate([edge_index[0], loop,
                             jnp.zeros((pad_e,), jnp.int32)]).reshape(-1, 128)
    dst2d = jnp.concatenate([edge_index[1], loop,
                             jnp.full((pad_e,), N, jnp.int32)]).reshape(-1, 128)
    one = jnp.ones((16,), jnp.float32)
    beta_b = jnp.broadcast_to(beta2.astype(jnp.float32), (16,))

    hs1, xn1 = _proj_norm(x_pad, W_emb)
    o1 = _conv(hs1, xn1, src2d, dst2d, one)
    xn2 = _norm_only(o1)
    o2 = _conv(o1, xn2, src2d, dst2d, beta_b)
    y = _decode(o2, W_dec)
    return y[:N]
